# baseline (speedup 1.0000x reference)
import functools

import jax
import jax.numpy as jnp
from jax.experimental import pallas as pl
from jax.experimental.pallas import tpu as pltpu

EPS = 1e-5


def _stats_kernel(x_ref, w_ref, sum_ref, sq_ref, acc_s, acc_q):
    j = pl.program_id(1)
    k = pl.program_id(2)
    nj = pl.num_programs(1)
    nk = pl.num_programs(2)

    @pl.when((j == 0) & (k == 0))
    def _():
        acc_s[...] = jnp.zeros_like(acc_s)
        acc_q[...] = jnp.zeros_like(acc_q)

    y = jax.lax.dot_general(
        w_ref[...], x_ref[0],
        dimension_numbers=(((1,), (0,)), ((), ())),
        preferred_element_type=jnp.float32)
    acc_s[...] += jnp.sum(y, axis=1, keepdims=True)
    acc_q[...] += jnp.sum(y * y, axis=1, keepdims=True)

    @pl.when((j == nj - 1) & (k == nk - 1))
    def _():
        sum_ref[0] = acc_s[...]
        sq_ref[0] = acc_q[...]


def _apply_kernel(x_ref, w_ref, sum_ref, sq_ref, g_ref, beta_ref, o_ref,
                  *, inv_n):
    s = sum_ref[0] + sum_ref[1]
    q = sq_ref[0] + sq_ref[1]
    mean = s * inv_n
    var = q * inv_n - mean * mean
    scale = g_ref[...] * jax.lax.rsqrt(var + EPS)
    shift = beta_ref[...] - mean * scale
    y = jax.lax.dot_general(
        w_ref[...], x_ref[0],
        dimension_numbers=(((1,), (0,)), ((), ())),
        preferred_element_type=jnp.float32)
    o_ref[0] = jnp.maximum(y * scale + shift, 0.0)


def _pick_tile(L):
    for tl in (2048, 1024, 512, 256, 128):
        if L % tl == 0:
            return tl
    return L


def kernel(x, w, b, g, beta):
    del b
    B, Cin, L = x.shape
    Cout = w.shape[0]

    xb = x.astype(jnp.bfloat16)
    wb = w.astype(jnp.bfloat16)
    g2 = g.reshape(Cout, 1).astype(jnp.float32)
    beta2 = beta.reshape(Cout, 1).astype(jnp.float32)
    inv_n = 1.0 / float(B * L)

    tl = _pick_tile(L)
    njl = L // tl
    bh = B // 2 if B % 2 == 0 else B
    ncore = 2 if B % 2 == 0 else 1

    sums, sqs = pl.pallas_call(
        _stats_kernel,
        out_shape=(jax.ShapeDtypeStruct((ncore, Cout, 1), jnp.float32),
                   jax.ShapeDtypeStruct((ncore, Cout, 1), jnp.float32)),
        grid=(ncore, bh, njl),
        in_specs=[
            pl.BlockSpec((1, Cin, tl), lambda i, j, k: (i * (B // 2) + j if B % 2 == 0 else j, 0, k)),
            pl.BlockSpec((Cout, Cin), lambda i, j, k: (0, 0)),
        ],
        out_specs=(pl.BlockSpec((1, Cout, 1), lambda i, j, k: (i, 0, 0)),
                   pl.BlockSpec((1, Cout, 1), lambda i, j, k: (i, 0, 0))),
        scratch_shapes=[pltpu.VMEM((Cout, 1), jnp.float32),
                        pltpu.VMEM((Cout, 1), jnp.float32)],
        compiler_params=pltpu.CompilerParams(
            dimension_semantics=("parallel", "arbitrary", "arbitrary"),
            vmem_limit_bytes=64 * 1024 * 1024),
    )(xb, wb)

    if ncore == 1:
        sums = jnp.concatenate([sums, jnp.zeros_like(sums)], axis=0)
        sqs = jnp.concatenate([sqs, jnp.zeros_like(sqs)], axis=0)

    out = pl.pallas_call(
        functools.partial(_apply_kernel, inv_n=inv_n),
        out_shape=jax.ShapeDtypeStruct((B, Cout, L), jnp.float32),
        grid=(B, njl),
        in_specs=[
            pl.BlockSpec((1, Cin, tl), lambda i, k: (i, 0, k)),
            pl.BlockSpec((Cout, Cin), lambda i, k: (0, 0)),
            pl.BlockSpec((2, Cout, 1), lambda i, k: (0, 0, 0)),
            pl.BlockSpec((2, Cout, 1), lambda i, k: (0, 0, 0)),
            pl.BlockSpec((Cout, 1), lambda i, k: (0, 0)),
            pl.BlockSpec((Cout, 1), lambda i, k: (0, 0)),
        ],
        out_specs=pl.BlockSpec((1, Cout, tl), lambda i, k: (i, 0, k)),
        compiler_params=pltpu.CompilerParams(
            dimension_semantics=("parallel", "parallel"),
            vmem_limit_bytes=64 * 1024 * 1024),
    )(xb, wb, sums, sqs, g2, beta2)

    return out


# speedup vs baseline: 1.3767x; 1.3767x over previous
import functools

import jax
import jax.numpy as jnp
from jax.experimental import pallas as pl
from jax.experimental.pallas import tpu as pltpu

EPS = 1e-5


def _fused_kernel(x_ref, w_ref, g_ref, beta_ref, o_ref, xb_ref, acc_s, acc_q,
                  *, n1, inv_n):
    t = pl.program_id(0)

    @pl.when(t == 0)
    def _():
        acc_s[...] = jnp.zeros_like(acc_s)
        acc_q[...] = jnp.zeros_like(acc_q)

    @pl.when(t < n1)
    def _():
        xb = x_ref[0].astype(jnp.bfloat16)
        xb_ref[t] = xb
        y = jax.lax.dot_general(
            w_ref[...], xb,
            dimension_numbers=(((1,), (0,)), ((), ())),
            preferred_element_type=jnp.float32)
        acc_s[...] += jnp.sum(y, axis=1, keepdims=True)
        acc_q[...] += jnp.sum(y * y, axis=1, keepdims=True)

    @pl.when(t >= n1)
    def _():
        mean = acc_s[...] * inv_n
        var = acc_q[...] * inv_n - mean * mean
        scale = g_ref[...] * jax.lax.rsqrt(var + EPS)
        shift = beta_ref[...] - mean * scale
        y = jax.lax.dot_general(
            w_ref[...], xb_ref[t - n1],
            dimension_numbers=(((1,), (0,)), ((), ())),
            preferred_element_type=jnp.float32)
        o_ref[0] = jnp.maximum(y * scale + shift, 0.0)


def _pick_tile(L):
    for tl in (2048, 1024, 512, 256, 128):
        if L % tl == 0:
            return tl
    return L


def kernel(x, w, b, g, beta):
    del b
    B, Cin, L = x.shape
    Cout = w.shape[0]

    wb = w.astype(jnp.bfloat16)
    g2 = g.reshape(Cout, 1).astype(jnp.float32)
    beta2 = beta.reshape(Cout, 1).astype(jnp.float32)
    inv_n = 1.0 / float(B * L)

    tl = _pick_tile(L)
    njl = L // tl
    n1 = B * njl

    def x_index(t):
        s = jnp.minimum(t, n1 - 1)
        return (s // njl, 0, s % njl)

    def o_index(t):
        u = jnp.maximum(t - n1, 0)
        return (u // njl, 0, u % njl)

    return pl.pallas_call(
        functools.partial(_fused_kernel, n1=n1, inv_n=inv_n),
        out_shape=jax.ShapeDtypeStruct((B, Cout, L), jnp.float32),
        grid=(2 * n1,),
        in_specs=[
            pl.BlockSpec((1, Cin, tl), x_index),
            pl.BlockSpec((Cout, Cin), lambda t: (0, 0)),
            pl.BlockSpec((Cout, 1), lambda t: (0, 0)),
            pl.BlockSpec((Cout, 1), lambda t: (0, 0)),
        ],
        out_specs=pl.BlockSpec((1, Cout, tl), o_index),
        scratch_shapes=[
            pltpu.VMEM((n1, Cin, tl), jnp.bfloat16),
            pltpu.VMEM((Cout, 1), jnp.float32),
            pltpu.VMEM((Cout, 1), jnp.float32),
        ],
        compiler_params=pltpu.CompilerParams(
            dimension_semantics=("arbitrary",),
            vmem_limit_bytes=56 * 1024 * 1024),
    )(x, wb, g2, beta2)


# speedup vs baseline: 1.9018x; 1.3814x over previous
import functools

import jax
import jax.numpy as jnp
from jax.experimental import pallas as pl
from jax.experimental.pallas import tpu as pltpu

EPS = 1e-5


def _fused_kernel(x_ref, w_ref, g_ref, beta_ref, o_ref, xb_ref, acc_s, acc_q,
                  *, n1, inv_n):
    t = pl.program_id(0)

    @pl.when(t == 0)
    def _():
        acc_s[...] = jnp.zeros_like(acc_s)
        acc_q[...] = jnp.zeros_like(acc_q)

    @pl.when(t < n1)
    def _():
        xb = x_ref[0].astype(jnp.bfloat16)
        xb_ref[t] = xb
        y = jax.lax.dot_general(
            w_ref[...], xb,
            dimension_numbers=(((1,), (0,)), ((), ())),
            preferred_element_type=jnp.float32)
        acc_s[...] += jnp.sum(y, axis=1, keepdims=True)
        acc_q[...] += jnp.sum(y * y, axis=1, keepdims=True)

    @pl.when(t >= n1)
    def _():
        mean = acc_s[...] * inv_n
        var = acc_q[...] * inv_n - mean * mean
        scale = g_ref[...] * jax.lax.rsqrt(var + EPS)
        shift = beta_ref[...] - mean * scale
        y = jax.lax.dot_general(
            w_ref[...], xb_ref[t - n1],
            dimension_numbers=(((1,), (0,)), ((), ())),
            preferred_element_type=jnp.float32)
        o_ref[0] = jnp.maximum(y * scale + shift, 0.0)


def _pick_tile(L):
    for tl in (4096, 2048, 1024, 512, 256, 128):
        if L % tl == 0:
            return tl
    return L


def kernel(x, w, b, g, beta):
    del b
    B, Cin, L = x.shape
    Cout = w.shape[0]

    wb = w.astype(jnp.bfloat16)
    g2 = g.reshape(Cout, 1).astype(jnp.float32)
    beta2 = beta.reshape(Cout, 1).astype(jnp.float32)
    inv_n = 1.0 / float(B * L)

    tl = _pick_tile(L)
    njl = L // tl
    n1 = B * njl

    def x_index(t):
        s = jnp.minimum(t, n1 - 1)
        return (s // njl, 0, s % njl)

    def o_index(t):
        u = jnp.maximum(t - n1, 0)
        return (u // njl, 0, u % njl)

    return pl.pallas_call(
        functools.partial(_fused_kernel, n1=n1, inv_n=inv_n),
        out_shape=jax.ShapeDtypeStruct((B, Cout, L), jnp.float32),
        grid=(2 * n1,),
        in_specs=[
            pl.BlockSpec((1, Cin, tl), x_index),
            pl.BlockSpec((Cout, Cin), lambda t: (0, 0)),
            pl.BlockSpec((Cout, 1), lambda t: (0, 0)),
            pl.BlockSpec((Cout, 1), lambda t: (0, 0)),
        ],
        out_specs=pl.BlockSpec((1, Cout, tl), o_index),
        scratch_shapes=[
            pltpu.VMEM((n1, Cin, tl), jnp.bfloat16),
            pltpu.VMEM((Cout, 1), jnp.float32),
            pltpu.VMEM((Cout, 1), jnp.float32),
        ],
        compiler_params=pltpu.CompilerParams(
            dimension_semantics=("arbitrary",),
            vmem_limit_bytes=56 * 1024 * 1024),
    )(x, wb, g2, beta2)
